# baseline (speedup 1.0000x reference)
import functools

import jax
import jax.numpy as jnp
from jax.experimental import pallas as pl
from jax.experimental.pallas import tpu as pltpu

_TOPK = 8
_NE = 64
_BLK = 512
_NEG = float("-inf")


def _router_kernel(x_ref, w_ref, b_ref, router_ref, idx_ref):
    x = x_ref[...]
    w = w_ref[...]
    logits_t = jax.lax.dot_general(
        w, x, (((1,), (1,)), ((), ())),
        preferred_element_type=jnp.float32)
    logits_t = logits_t + b_ref[...]

    iota_e = jax.lax.broadcasted_iota(jnp.int32, logits_t.shape, 0)
    work = logits_t
    mask = jnp.zeros(logits_t.shape, jnp.bool_)
    idx_rows = []
    maxv = None
    for k in range(_TOPK):
        m = jnp.max(work, axis=0, keepdims=True)
        if k == 0:
            maxv = m
        is_m = work == m
        idx = jnp.min(jnp.where(is_m, iota_e, _NE), axis=0,
                      keepdims=True)
        sel = iota_e == idx
        idx_rows.append(idx)
        mask = jnp.logical_or(mask, sel)
        work = jnp.where(sel, _NEG, work)

    ex = jnp.where(mask, jnp.exp(logits_t - maxv), jnp.float32(0.0))
    denom = jnp.sum(ex, axis=0, keepdims=True)
    router_t = ex / denom
    router_ref[...] = router_t.T
    idx_t = jnp.concatenate(idx_rows, axis=0)
    idx_ref[...] = idx_t.T


@jax.jit
def kernel(x, W, b):
    n_rows = x.shape[0]
    grid = (n_rows // _BLK,)
    router, idx = pl.pallas_call(
        _router_kernel,
        grid=grid,
        in_specs=[
            pl.BlockSpec((_BLK, x.shape[1]), lambda i: (i, 0)),
            pl.BlockSpec((_NE, x.shape[1]), lambda i: (0, 0)),
            pl.BlockSpec((_NE, 1), lambda i: (0, 0)),
        ],
        out_specs=[
            pl.BlockSpec((_BLK, _NE), lambda i: (i, 0)),
            pl.BlockSpec((_BLK, _TOPK), lambda i: (i, 0)),
        ],
        out_shape=[
            jax.ShapeDtypeStruct((n_rows, _NE), jnp.float32),
            jax.ShapeDtypeStruct((n_rows, _TOPK), jnp.int32),
        ],
        compiler_params=pltpu.CompilerParams(
            dimension_semantics=("parallel",)),
    )(x, W, b.reshape(_NE, 1))
    return router, idx


# speedup vs baseline: 1.0140x; 1.0140x over previous
import jax
import jax.numpy as jnp
from jax.experimental import pallas as pl
from jax.experimental.pallas import tpu as pltpu

_TOPK = 8
_NE = 64
_BLK = 512


def _router_kernel(x_ref, w_ref, b_ref, router_ref, idx_ref):
    x = x_ref[...]
    w = w_ref[...]
    logits_t = jax.lax.dot_general(
        w, x, (((1,), (1,)), ((), ())),
        preferred_element_type=jnp.float32)
    logits_t = logits_t + b_ref[...]

    raw = jax.lax.bitcast_convert_type(logits_t, jnp.int32)
    mono = raw ^ jax.lax.shift_right_logical(
        jax.lax.shift_right_arithmetic(raw, 31), 1)
    iota_e = jax.lax.broadcasted_iota(jnp.int32, logits_t.shape, 0)

    def argmax_rows(v, i):
        while v.shape[0] > 1:
            h = v.shape[0] // 2
            take_a = v[:h] >= v[h:]
            v = jnp.where(take_a, v[:h], v[h:])
            i = jnp.where(take_a, i[:h], i[h:])
        return v, i

    work = mono
    mask = jnp.zeros(logits_t.shape, jnp.bool_)
    idx_rows = []
    m0 = None
    for k in range(_TOPK):
        m, mi = argmax_rows(work, iota_e)
        if k == 0:
            m0 = m
        sel = iota_e == mi
        mask = jnp.logical_or(mask, sel)
        work = jnp.where(sel, jnp.int32(-2**31), work)
        idx_rows.append(mi)

    maxv = jax.lax.bitcast_convert_type(
        m0 ^ jax.lax.shift_right_logical(
            jax.lax.shift_right_arithmetic(m0, 31), 1), jnp.float32)
    ex = jnp.where(mask, jnp.exp(logits_t - maxv), jnp.float32(0.0))
    denom = jnp.sum(ex, axis=0, keepdims=True)
    router_t = ex / denom
    router_ref[...] = router_t.T
    idx_t = jnp.concatenate(idx_rows, axis=0)
    idx_ref[...] = idx_t.T


@jax.jit
def kernel(x, W, b):
    n_rows = x.shape[0]
    grid = (n_rows // _BLK,)
    router, idx = pl.pallas_call(
        _router_kernel,
        grid=grid,
        in_specs=[
            pl.BlockSpec((_BLK, x.shape[1]), lambda i: (i, 0)),
            pl.BlockSpec((_NE, x.shape[1]), lambda i: (0, 0)),
            pl.BlockSpec((_NE, 1), lambda i: (0, 0)),
        ],
        out_specs=[
            pl.BlockSpec((_BLK, _NE), lambda i: (i, 0)),
            pl.BlockSpec((_BLK, _TOPK), lambda i: (i, 0)),
        ],
        out_shape=[
            jax.ShapeDtypeStruct((n_rows, _NE), jnp.float32),
            jax.ShapeDtypeStruct((n_rows, _TOPK), jnp.int32),
        ],
        compiler_params=pltpu.CompilerParams(
            dimension_semantics=("parallel",)),
    )(x, W, b.reshape(_NE, 1))
    return router, idx


# speedup vs baseline: 1.0156x; 1.0015x over previous
import jax
import jax.numpy as jnp
from jax.experimental import pallas as pl
from jax.experimental.pallas import tpu as pltpu

_TOPK = 8
_NE = 64
_BLK = 512


def _router_kernel(x_ref, w_ref, b_ref, router_ref, idx_ref):
    x = x_ref[...]
    w = w_ref[...]
    logits_t = jax.lax.dot_general(
        w, x, (((1,), (1,)), ((), ())),
        preferred_element_type=jnp.float32)
    logits_t = logits_t + b_ref[...]

    raw = jax.lax.bitcast_convert_type(logits_t, jnp.int32)
    mono = raw ^ jax.lax.shift_right_logical(
        jax.lax.shift_right_arithmetic(raw, 31), 1)
    iota_e = jax.lax.broadcasted_iota(jnp.int32, logits_t.shape, 0)

    def argmax_rows(v, i):
        while v.shape[0] > 1:
            h = v.shape[0] // 2
            take_a = v[:h] >= v[h:]
            v = jnp.where(take_a, v[:h], v[h:])
            i = jnp.where(take_a, i[:h], i[h:])
        return v, i

    work = mono
    idx_rows = []
    m0 = mlast = None
    for k in range(_TOPK):
        m, mi = argmax_rows(work, iota_e)
        if k == 0:
            m0 = m
        mlast = m
        if k + 1 < _TOPK:
            sel = iota_e == mi
            work = jnp.where(sel, jnp.int32(-2**31), work)
        idx_rows.append(mi)
    mask = mono >= mlast

    maxv = jax.lax.bitcast_convert_type(
        m0 ^ jax.lax.shift_right_logical(
            jax.lax.shift_right_arithmetic(m0, 31), 1), jnp.float32)
    ex = jnp.where(mask, jnp.exp(logits_t - maxv), jnp.float32(0.0))
    denom = jnp.sum(ex, axis=0, keepdims=True)
    router_t = ex / denom
    router_ref[...] = router_t.T
    idx_t = jnp.concatenate(idx_rows, axis=0)
    idx_ref[...] = idx_t.T


@jax.jit
def kernel(x, W, b):
    n_rows = x.shape[0]
    grid = (n_rows // _BLK,)
    router, idx = pl.pallas_call(
        _router_kernel,
        grid=grid,
        in_specs=[
            pl.BlockSpec((_BLK, x.shape[1]), lambda i: (i, 0)),
            pl.BlockSpec((_NE, x.shape[1]), lambda i: (0, 0)),
            pl.BlockSpec((_NE, _BLK), lambda i: (0, 0)),
        ],
        out_specs=[
            pl.BlockSpec((_BLK, _NE), lambda i: (i, 0)),
            pl.BlockSpec((_BLK, _TOPK), lambda i: (i, 0)),
        ],
        out_shape=[
            jax.ShapeDtypeStruct((n_rows, _NE), jnp.float32),
            jax.ShapeDtypeStruct((n_rows, _TOPK), jnp.int32),
        ],
        compiler_params=pltpu.CompilerParams(
            dimension_semantics=("parallel",)),
    )(x, W, jnp.broadcast_to(b[:, None], (_NE, _BLK)))
    return router, idx
